# Initial kernel scaffold; baseline (speedup 1.0000x reference)
import jax
import jax.numpy as jnp
from jax.experimental import pallas as pl


def kernel(all_ts, W, cumhisto):
    raise NotImplementedError("write your pallas kernel here")


# speedup vs baseline: 11.6937x; 11.6937x over previous
import jax
import jax.numpy as jnp
from jax.experimental import pallas as pl
from jax.experimental.pallas import tpu as pltpu

_N_EVENTS = 4096
_N_NEURONS = 1024
_TS = 256


def _body(all_ts_ref, w_in_ref, ch_in_ref, out_ref, w_ref, ch_ref):
    w_ref[...] = w_in_ref[...]
    ch_ref[...] = ch_in_ref[...]
    chsum0 = jnp.sum(ch_in_ref[...])
    iota_n = jax.lax.iota(jnp.int32, _N_NEURONS)

    def step(t, carry):
        ts = all_ts_ref[pl.ds(t, 1), :]
        s = jnp.sqrt(jnp.sum(ts * ts))
        tsd = ts / s
        w = w_ref[...]
        mv = jnp.sum(w * tsd, axis=1)
        wn2 = jnp.sum(w * w, axis=1)
        beta = mv * jax.lax.rsqrt(wn2)
        ch = ch_ref[...]
        chsum = chsum0 + t.astype(jnp.float32)
        gain = jnp.exp((1.0 - (ch * 1024.0) / chsum) * 0.25)
        bh = gain * beta
        n = jnp.argmax(bh).astype(jnp.int32)
        onehot = iota_n == n
        ch_n = jnp.sum(jnp.where(onehot, ch, 0.0))
        beta_n = jnp.sum(jnp.where(onehot, beta, 0.0))
        alpha = jnp.float32(0.01) / (1.0 + ch_n * jnp.float32(5e-5))
        a = alpha * beta_n
        ck = w_ref[pl.ds(n, 1), :]
        w_ref[pl.ds(n, 1), :] = ck + a * (tsd - ck)
        ch_ref[...] = jnp.where(onehot, ch + 1.0, ch)
        out_ref[0] = n
        return carry

    jax.lax.fori_loop(0, _N_EVENTS, step, jnp.int32(0))


def kernel(all_ts, W, cumhisto):
    out = pl.pallas_call(
        _body,
        out_shape=jax.ShapeDtypeStruct((1,), jnp.int32),
        in_specs=[
            pl.BlockSpec(memory_space=pltpu.VMEM),
            pl.BlockSpec(memory_space=pltpu.VMEM),
            pl.BlockSpec(memory_space=pltpu.VMEM),
        ],
        out_specs=pl.BlockSpec(memory_space=pltpu.SMEM),
        scratch_shapes=[
            pltpu.VMEM((_N_NEURONS, _TS), jnp.float32),
            pltpu.VMEM((_N_NEURONS,), jnp.float32),
        ],
    )(all_ts, W, cumhisto)
    return out[0]
